# baseline (speedup 1.0000x reference)
import functools

import jax
import jax.numpy as jnp
from jax import lax
from jax.experimental import pallas as pl
from jax.experimental.pallas import tpu as pltpu
from jax.experimental.pallas import tpu_sc as plsc

B = 16384
D = 128
N = 100000
K = B // 2
NC = 2
NS = 16
NW = NC * NS
CHUNK = B // NW
PAD = 64
TAIL = N - B
TAIL_PER = 2616


def _iota16():
    return lax.iota(jnp.int32, 16)


def _stat_tc(grad):
    def body(g_ref, o_ref):
        x = g_ref[...]
        ss = jnp.sum(x * x, axis=1)
        o_ref[...] = lax.bitcast_convert_type(
            jnp.sqrt(ss).reshape(16, 128), jnp.int32)

    return pl.pallas_call(
        body,
        grid=(8,),
        in_specs=[pl.BlockSpec((2048, 128), lambda i: (i, 0))],
        out_specs=pl.BlockSpec((16, 128), lambda i: (i, 0)),
        out_shape=jax.ShapeDtypeStruct((128, 128), jnp.int32),
    )(grad)


def _select_body(bits_hbm, lid_hbm, pos_hbm, time_hbm,
                 stat_v, hist2_v, hist_v, cum_v, counts_v,
                 posb_v, timeb_v, lidv_v, lidx4_v, hidx8_v,
                 cidx_v, cval_v, filln_v, fillz_v, zero_v,
                 hist_g, counts_g):
    c = lax.axis_index("c")
    s = lax.axis_index("s")
    own = c * NS + s
    iota = _iota16()
    ones = jnp.full((16,), 1, jnp.int32)
    zeros = jnp.full((16,), 0, jnp.int32)

    def _fill(i, _):
        zero_v[pl.ds(i * 16, 16)] = zeros
        return 0
    lax.fori_loop(0, 68, _fill, 0)

    def _fill2(i, _):
        filln_v[pl.ds(i * 16, 16)] = jnp.full((16,), -1, jnp.int32)
        fillz_v[pl.ds(i * 16, 16)] = zeros
        return 0
    lax.fori_loop(0, 164, _fill2, 0)

    def _zh(i, _):
        hist2_v[pl.ds(i * 16, 16)] = zeros
        return 0
    lax.fori_loop(0, 256, _zh, 0)

    for p in range(4):
        for r in range(2):
            for g in range(8):
                hidx8_v[2 * p + r, pl.ds(g * 16, 16)] = (
                    p * 256 + r * 128 + g * 16 + iota)

    @pl.when(s == 0)
    def _zero_shared():
        pltpu.sync_copy(zero_v.at[pl.ds(0, 1024)], hist_g)
        pltpu.sync_copy(zero_v.at[pl.ds(0, 64)], counts_g)

    pltpu.sync_copy(bits_hbm.at[pl.ds(s * CHUNK, CHUNK)],
                    stat_v.at[pl.ds(0, CHUNK)])
    pltpu.sync_copy(bits_hbm.at[pl.ds((s + NS) * CHUNK, CHUNK)],
                    stat_v.at[pl.ds(CHUNK, CHUNK)])

    plsc.subcore_barrier()

    prefix = jnp.int32(0)
    k_rem = jnp.int32(K)
    for p in range(4):
        shift = 24 - 8 * p

        def _hist(i, _):
            v = stat_v[pl.ds(i * 16, 16)]
            b = lax.shift_right_logical(v, shift) & 0xFF
            if p == 0:
                m = None
            else:
                m = lax.shift_right_logical(v, shift + 8) == lax.shift_right_logical(
                    prefix, shift + 8)
            plsc.addupdate_scatter(hist2_v, [iota * 256 + b], ones, mask=m)
            return 0
        lax.fori_loop(0, 64, _hist, 0)

        def _reduce(g, _):
            acc = hist2_v[pl.ds(g * 16, 16)]
            hist2_v[pl.ds(g * 16, 16)] = zeros
            for r in range(1, 16):
                acc = acc + hist2_v[pl.ds(r * 256 + g * 16, 16)]
                hist2_v[pl.ds(r * 256 + g * 16, 16)] = zeros
            hist_v[pl.ds(g * 16, 16)] = acc
            return 0
        lax.fori_loop(0, 16, _reduce, 0)

        for r in range(2):
            pltpu.sync_copy(hist_v.at[pl.ds(r * 128, 128)],
                            hist_g.at[hidx8_v.at[2 * p + r]], add=True)
        plsc.subcore_barrier()

        pltpu.sync_copy(hist_g.at[pl.ds(p * 256, 256)], hist_v)

        def _cum(g, carry):
            v = hist_v[pl.ds(g * 16, 16)]
            cv = plsc.cumsum(v) + carry
            cum_v[pl.ds(g * 16, 16)] = cv
            return carry + jnp.sum(v)
        lax.fori_loop(0, 16, _cum, jnp.int32(0))

        def _bcnt(g, acc):
            cv = cum_v[pl.ds(g * 16, 16)]
            return acc + jnp.sum(jnp.where(cv < k_rem, 1, 0))
        b = lax.fori_loop(0, 16, _bcnt, jnp.int32(0))

        def _below(g, acc):
            cv = cum_v[pl.ds(g * 16, 16)]
            gl = g * 16 + iota
            return acc + jnp.sum(jnp.where(gl == b - 1, cv, 0))
        below = lax.fori_loop(0, 16, _below, jnp.int32(0))

        k_rem = k_rem - below
        prefix = prefix | lax.shift_left(b, shift)

    t_bits = prefix

    cnts = []
    for h in range(2):
        def _cnt(g, acc):
            v = stat_v[pl.ds(h * CHUNK + g * 16, 16)]
            return acc + jnp.sum(jnp.where(v <= t_bits, 1, 0))
        cnts.append(lax.fori_loop(0, 32, _cnt, jnp.int32(0)))

    cidx_v[0, :] = jnp.where(iota == 0, s,
                             jnp.where(iota == 1, s + NS, 32 + iota))
    cval_v[...] = jnp.where(iota == 0, cnts[0],
                            jnp.where(iota == 1, cnts[1], 0))
    pltpu.sync_copy(cval_v, counts_g.at[cidx_v.at[0]], add=True)
    plsc.subcore_barrier()
    pltpu.sync_copy(counts_g.at[pl.ds(0, 32)], counts_v)

    def _pfx(g, acc):
        v = counts_v[pl.ds(g * 16, 16)]
        gl = g * 16 + iota
        return acc + jnp.sum(jnp.where(gl < own, v, 0))
    prefix_own = lax.fori_loop(0, 2, _pfx, jnp.int32(0))

    base = own * CHUNK
    dump = jnp.int32(K) + 2 * own
    run = prefix_own
    for g in range(32):
        v = stat_v[pl.ds(c * CHUNK + g * 16, 16)]
        m = v <= t_bits
        mi = jnp.where(m, 1, 0)
        incl = plsc.cumsum(mi)
        grank = run + incl - mi
        run = run + jnp.sum(mi)
        sel = m & (grank < K)
        posb_v[pl.ds(g * 16, 16)] = jnp.where(sel, grank, -1)
        timeb_v[pl.ds(g * 16, 16)] = jnp.where(sel, 1, 0)
        lidv_v[pl.ds(g * 16, 16)] = base + g * 16 + iota
        lidx4_v[g // 8, pl.ds((g % 8) * 16, 16)] = jnp.where(sel, grank, dump)

    pltpu.sync_copy(posb_v, pos_hbm.at[pl.ds(base, CHUNK)])
    pltpu.sync_copy(timeb_v, time_hbm.at[pl.ds(base, CHUNK)])
    for r in range(4):
        pltpu.sync_copy(lidv_v.at[pl.ds(r * 128, 128)],
                        lid_hbm.at[lidx4_v.at[r]])

    tbase = jnp.minimum(B + own * TAIL_PER, N - TAIL_PER)
    pltpu.sync_copy(filln_v.at[pl.ds(0, TAIL_PER)],
                    pos_hbm.at[pl.ds(tbase, TAIL_PER)])
    pltpu.sync_copy(fillz_v.at[pl.ds(0, TAIL_PER)],
                    time_hbm.at[pl.ds(tbase, TAIL_PER)])


def _select_sc(bits):
    mesh = plsc.VectorSubcoreMesh(core_axis_name="c", subcore_axis_name="s",
                                  num_cores=NC, num_subcores=NS)
    return pl.kernel(
        _select_body,
        out_type=(
            jax.ShapeDtypeStruct((K + PAD,), jnp.int32),
            jax.ShapeDtypeStruct((N,), jnp.int32),
            jax.ShapeDtypeStruct((N,), jnp.int32),
        ),
        mesh=mesh,
        compiler_params=pltpu.CompilerParams(needs_layout_passes=False),
        scratch_types=[
            pltpu.VMEM((2 * CHUNK,), jnp.int32),
            pltpu.VMEM((16 * 256,), jnp.int32),
            pltpu.VMEM((256,), jnp.int32),
            pltpu.VMEM((256,), jnp.int32),
            pltpu.VMEM((32,), jnp.int32),
            pltpu.VMEM((CHUNK,), jnp.int32),
            pltpu.VMEM((CHUNK,), jnp.int32),
            pltpu.VMEM((CHUNK,), jnp.int32),
            pltpu.VMEM((4, 128), jnp.int32),
            pltpu.VMEM((8, 128), jnp.int32),
            pltpu.VMEM((1, 16), jnp.int32),
            pltpu.VMEM((16,), jnp.int32),
            pltpu.VMEM((2624,), jnp.int32),
            pltpu.VMEM((2624,), jnp.int32),
            pltpu.VMEM((1088,), jnp.int32),
            pltpu.VMEM_SHARED((1024,), jnp.int32),
            pltpu.VMEM_SHARED((64,), jnp.int32),
        ],
    )(bits)


def _gather_body(feats_hbm, lid_hbm, emb_hbm, idx2_v, rows_v, sem0, sem1):
    c = lax.axis_index("c")
    s = lax.axis_index("s")
    wid = c * NS + s
    base = wid * 256
    for r in range(2):
        pltpu.sync_copy(lid_hbm.at[pl.ds(base + r * 128, 128)],
                        idx2_v.at[r])
    cp0 = pltpu.async_copy(feats_hbm.at[idx2_v.at[0]], rows_v.at[0], sem0)
    cp1 = pltpu.async_copy(feats_hbm.at[idx2_v.at[1]], rows_v.at[1], sem1)
    cp0.wait()
    pltpu.sync_copy(rows_v.at[0], emb_hbm.at[pl.ds(base, 128)])
    cp1.wait()
    pltpu.sync_copy(rows_v.at[1], emb_hbm.at[pl.ds(base + 128, 128)])


def _gather_sc(feats, lid_pad):
    mesh = plsc.VectorSubcoreMesh(core_axis_name="c", subcore_axis_name="s",
                                  num_cores=NC, num_subcores=NS)
    return pl.kernel(
        _gather_body,
        out_type=jax.ShapeDtypeStruct((K, D), jnp.float32),
        mesh=mesh,
        scratch_types=[
            pltpu.VMEM((2, 128), jnp.int32),
            pltpu.VMEM((2, 128, D), jnp.float32),
            pltpu.SemaphoreType.DMA,
            pltpu.SemaphoreType.DMA,
        ],
    )(feats, lid_pad)


def kernel(gids, feats, grad, pos, time, emb, index_to_gid):
    bits = _stat_tc(grad).reshape(B)
    lid_pad, pos_o, time_o = _select_sc(bits)
    emb_o = _gather_sc(feats, lid_pad)
    return emb_o, pos_o, lid_pad[:K], time_o


# speedup vs baseline: 13.8604x; 13.8604x over previous
import jax
import jax.numpy as jnp
from jax import lax
from jax.experimental import pallas as pl
from jax.experimental.pallas import tpu as pltpu
from jax.experimental.pallas import tpu_sc as plsc

B = 16384
D = 128
N = 100000
K = B // 2
NC = 2
NS = 16
NW = NC * NS
WIN = K // NW
PR = (N + D - 1) // D
SEG_ROWS = B // D // NW
TAIL_ROWS = 21
NG = B // 16


def _stat_tc(grad):
    def body(g_ref, o_ref):
        x = g_ref[...]
        ss = jnp.sum(x * x, axis=1)
        o_ref[...] = lax.bitcast_convert_type(
            jnp.sqrt(ss).reshape(16, 128), jnp.int32)

    return pl.pallas_call(
        body,
        grid=(8,),
        in_specs=[pl.BlockSpec((2048, 128), lambda i: (i, 0))],
        out_specs=pl.BlockSpec((16, 128), lambda i: (i, 0)),
        out_shape=jax.ShapeDtypeStruct((128, 128), jnp.int32),
    )(grad)


def _fused_body(bits_hbm, feats_hbm, emb_hbm, lid_hbm, pos_hbm, time_hbm,
                stat_v, hist2_v, hist_v, cum_v, win_v, win2_v, rows_v,
                posb_v, timeb_v, filln_v, fillz_v, sem0, sem1):
    c = lax.axis_index("c")
    s = lax.axis_index("s")
    w = c * NS + s
    iota = lax.iota(jnp.int32, 16)
    ones = jnp.full((16,), 1, jnp.int32)
    zeros = jnp.full((16,), 0, jnp.int32)

    with jax.named_scope("ph_fill"):
        def _zh(i, _):
            hist2_v[pl.ds(i * 16, 16)] = zeros
            return 0
        lax.fori_loop(0, 256, _zh, 0)

        neg1 = jnp.full((16,), -1, jnp.int32)
        for r in range(TAIL_ROWS):
            for j in range(8):
                filln_v[r, pl.ds(j * 16, 16)] = neg1
                fillz_v[r, pl.ds(j * 16, 16)] = zeros

    with jax.named_scope("ph_stage"):
        pltpu.sync_copy(bits_hbm, stat_v)

    prefix = jnp.int32(0)
    k_rem = jnp.int32(K)
    for p in range(4):
      with jax.named_scope(f"ph_radix{p}"):
        shift = 24 - 8 * p

        def _hist(i, _):
            v = stat_v[pl.ds(i * 16, 16)]
            b = lax.shift_right_logical(v, shift) & 0xFF
            if p == 0:
                m = None
            else:
                m = lax.shift_right_logical(v, shift + 8) == lax.shift_right_logical(
                    prefix, shift + 8)
            plsc.addupdate_scatter(hist2_v, [iota * 256 + b], ones, mask=m)
            return 0
        lax.fori_loop(0, NG, _hist, 0)

        def _reduce(g, _):
            acc = hist2_v[pl.ds(g * 16, 16)]
            hist2_v[pl.ds(g * 16, 16)] = zeros
            for r in range(1, 16):
                acc = acc + hist2_v[pl.ds(r * 256 + g * 16, 16)]
                hist2_v[pl.ds(r * 256 + g * 16, 16)] = zeros
            hist_v[pl.ds(g * 16, 16)] = acc
            return 0
        lax.fori_loop(0, 16, _reduce, 0)

        def _cum(g, carry):
            v = hist_v[pl.ds(g * 16, 16)]
            cv = plsc.cumsum(v) + carry
            cum_v[pl.ds(g * 16, 16)] = cv
            return carry + jnp.sum(v)
        lax.fori_loop(0, 16, _cum, jnp.int32(0))

        def _bcnt(g, acc):
            cv = cum_v[pl.ds(g * 16, 16)]
            return acc + jnp.sum(jnp.where(cv < k_rem, 1, 0))
        b = lax.fori_loop(0, 16, _bcnt, jnp.int32(0))

        def _below(g, acc):
            cv = cum_v[pl.ds(g * 16, 16)]
            gl = g * 16 + iota
            return acc + jnp.sum(jnp.where(gl == b - 1, cv, 0))
        below = lax.fori_loop(0, 16, _below, jnp.int32(0))

        k_rem = k_rem - below
        prefix = prefix | lax.shift_left(b, shift)

    t_bits = prefix

    with jax.named_scope("ph_scan"):
        own_g = w * (NG // NW)
        wlo = w * WIN

        def _scan(g, carry):
            run, pref_own = carry
            v = stat_v[pl.ds(g * 16, 16)]
            m = v <= t_bits
            mi = jnp.where(m, 1, 0)
            incl = plsc.cumsum(mi)
            grank = run + incl - mi
            inwin = m & (grank >= wlo) & (grank < wlo + WIN)
            plsc.store_scatter(win_v, [grank - wlo], g * 16 + iota,
                               mask=inwin)
            pref_own = jnp.where(g == own_g, run, pref_own)
            return run + jnp.sum(mi), pref_own
        _, pref_own = lax.fori_loop(0, NG, _scan, (jnp.int32(0), jnp.int32(0)))

    with jax.named_scope("ph_out"):
        run2 = pref_own
        for g in range(32):
            v = stat_v[pl.ds(w * (B // NW) + g * 16, 16)]
            m = v <= t_bits
            mi = jnp.where(m, 1, 0)
            incl = plsc.cumsum(mi)
            grank = run2 + incl - mi
            run2 = run2 + jnp.sum(mi)
            sel = m & (grank < K)
            posb_v[g // 8, pl.ds((g % 8) * 16, 16)] = jnp.where(sel, grank, -1)
            timeb_v[g // 8, pl.ds((g % 8) * 16, 16)] = jnp.where(sel, 1, 0)

        pltpu.sync_copy(posb_v, pos_hbm.at[pl.ds(w * SEG_ROWS, SEG_ROWS)])
        pltpu.sync_copy(timeb_v, time_hbm.at[pl.ds(w * SEG_ROWS, SEG_ROWS)])

    with jax.named_scope("ph_tail"):
        tbase = jnp.minimum(B // D + w * TAIL_ROWS, PR - TAIL_ROWS)
        pltpu.sync_copy(filln_v, pos_hbm.at[pl.ds(tbase, TAIL_ROWS)])
        pltpu.sync_copy(fillz_v, time_hbm.at[pl.ds(tbase, TAIL_ROWS)])

    with jax.named_scope("ph_gather"):
        for r in range(2):
            for j in range(8):
                win2_v[r, pl.ds(j * 16, 16)] = win_v[pl.ds(r * 128 + j * 16, 16)]
        pltpu.sync_copy(win2_v, lid_hbm.at[pl.ds(2 * w, 2)])
        cp0 = pltpu.async_copy(feats_hbm.at[win2_v.at[0]], rows_v.at[0], sem0)
        cp1 = pltpu.async_copy(feats_hbm.at[win2_v.at[1]], rows_v.at[1], sem1)
        cp0.wait()
        pltpu.sync_copy(rows_v.at[0], emb_hbm.at[pl.ds(wlo, 128)])
        cp1.wait()
        pltpu.sync_copy(rows_v.at[1], emb_hbm.at[pl.ds(wlo + 128, 128)])


def _fused_sc(bits, feats):
    mesh = plsc.VectorSubcoreMesh(core_axis_name="c", subcore_axis_name="s",
                                  num_cores=NC, num_subcores=NS)
    return pl.kernel(
        _fused_body,
        out_type=(
            jax.ShapeDtypeStruct((K, D), jnp.float32),
            jax.ShapeDtypeStruct((K // D, D), jnp.int32),
            jax.ShapeDtypeStruct((PR, D), jnp.int32),
            jax.ShapeDtypeStruct((PR, D), jnp.int32),
        ),
        mesh=mesh,
        compiler_params=pltpu.CompilerParams(needs_layout_passes=False,
                                             use_tc_tiling_on_sc=False),
        scratch_types=[
            pltpu.VMEM((B,), jnp.int32),
            pltpu.VMEM((16 * 256,), jnp.int32),
            pltpu.VMEM((256,), jnp.int32),
            pltpu.VMEM((256,), jnp.int32),
            pltpu.VMEM((WIN,), jnp.int32),
            pltpu.VMEM((2, 128), jnp.int32),
            pltpu.VMEM((2, 128, D), jnp.float32),
            pltpu.VMEM((SEG_ROWS, D), jnp.int32),
            pltpu.VMEM((SEG_ROWS, D), jnp.int32),
            pltpu.VMEM((TAIL_ROWS, D), jnp.int32),
            pltpu.VMEM((TAIL_ROWS, D), jnp.int32),
            pltpu.SemaphoreType.DMA,
            pltpu.SemaphoreType.DMA,
        ],
    )(bits, feats)


def kernel(gids, feats, grad, pos, time, emb, index_to_gid):
    bits = _stat_tc(grad).reshape(B)
    emb_o, lid2, pos2, time2 = _fused_sc(bits, feats)
    return (emb_o, pos2.reshape(-1)[:N], lid2.reshape(-1),
            time2.reshape(-1)[:N])


# speedup vs baseline: 15.7222x; 1.1343x over previous
import jax
import jax.numpy as jnp
from jax import lax
from jax.experimental import pallas as pl
from jax.experimental.pallas import tpu as pltpu
from jax.experimental.pallas import tpu_sc as plsc

B = 16384
D = 128
N = 100000
K = B // 2
NC = 2
NS = 16
NW = NC * NS
WIN = K // NW
PR = (N + D - 1) // D
SEG_ROWS = B // D // NW
TAIL_ROWS = 21
NG = B // 16


def _stat_tc(grad):
    def body(g_ref, o_ref):
        x = g_ref[...]
        ss = jnp.sum(x * x, axis=1)
        o_ref[...] = lax.bitcast_convert_type(
            jnp.sqrt(ss).reshape(16, 128), jnp.int32)

    return pl.pallas_call(
        body,
        grid=(8,),
        in_specs=[pl.BlockSpec((2048, 128), lambda i: (i, 0))],
        out_specs=pl.BlockSpec((16, 128), lambda i: (i, 0)),
        out_shape=jax.ShapeDtypeStruct((128, 128), jnp.int32),
    )(grad)


def _fused_body(bits_hbm, feats_hbm, emb_hbm, lid_hbm, pos_hbm, time_hbm,
                stat_v, hist2_v, hist_v, cum_v, cbuf_v, win_v, win2_v,
                rows_v, posb_v, timeb_v, filln_v, fillz_v, sem0, sem1):
    c = lax.axis_index("c")
    s = lax.axis_index("s")
    w = c * NS + s
    iota = lax.iota(jnp.int32, 16)
    ones = jnp.full((16,), 1, jnp.int32)
    zeros = jnp.full((16,), 0, jnp.int32)

    with jax.named_scope("ph_fill"):
        def _zh(i, _):
            hist2_v[pl.ds(i * 16, 16)] = zeros
            return 0
        lax.fori_loop(0, 512, _zh, 0)

        neg1 = jnp.full((16,), -1, jnp.int32)
        for r in range(TAIL_ROWS):
            for j in range(8):
                filln_v[r, pl.ds(j * 16, 16)] = neg1
                fillz_v[r, pl.ds(j * 16, 16)] = zeros

    with jax.named_scope("ph_stage"):
        pltpu.sync_copy(bits_hbm, stat_v)

    def _digit(hist_lo, k_rem):
        def _reduce(g, _):
            acc = hist2_v[pl.ds(g * 16, 16)]
            hist2_v[pl.ds(g * 16, 16)] = zeros
            for r in range(1, 32):
                acc = acc + hist2_v[pl.ds(r * 256 + g * 16, 16)]
                hist2_v[pl.ds(r * 256 + g * 16, 16)] = zeros
            hist_v[pl.ds(g * 16, 16)] = acc
            return 0
        lax.fori_loop(0, 16, _reduce, 0)

        def _cum(g, carry):
            v = hist_v[pl.ds(g * 16, 16)]
            cv = plsc.cumsum(v) + carry
            cum_v[pl.ds(g * 16, 16)] = cv
            return carry + jnp.sum(v)
        lax.fori_loop(0, 16, _cum, jnp.int32(0))

        def _bcnt(g, acc):
            cv = cum_v[pl.ds(g * 16, 16)]
            return acc + jnp.sum(jnp.where(cv < k_rem, 1, 0))
        b = lax.fori_loop(0, 16, _bcnt, jnp.int32(0))

        def _below(g, acc):
            cv = cum_v[pl.ds(g * 16, 16)]
            gl = g * 16 + iota
            return acc + jnp.sum(jnp.where(gl == b - 1, cv, 0))
        below = lax.fori_loop(0, 16, _below, jnp.int32(0))
        return b, below

    prefix = jnp.int32(0)
    k_rem = jnp.int32(K)
    for p in range(2):
      with jax.named_scope(f"ph_radix{p}"):
        shift = 24 - 8 * p

        def _hist(i, _):
            for half in range(2):
                v = stat_v[pl.ds((2 * i + half) * 16, 16)]
                b = lax.shift_right_logical(v, shift) & 0xFF
                if p == 0:
                    m = None
                else:
                    m = (lax.shift_right_logical(v, shift + 8) ==
                         lax.shift_right_logical(prefix, shift + 8))
                plsc.addupdate_scatter(
                    hist2_v, [(iota + 16 * half) * 256 + b], ones, mask=m)
            return 0
        lax.fori_loop(0, NG // 2, _hist, 0)
        b, below = _digit(hist_v, k_rem)
        k_rem = k_rem - below
        prefix = prefix | lax.shift_left(b, shift)

    with jax.named_scope("ph_radix2"):
        pref16 = lax.shift_right_logical(prefix, 16)

        def _hist2(i, carry):
            off = carry
            for half in range(2):
                v = stat_v[pl.ds((2 * i + half) * 16, 16)]
                m = lax.shift_right_logical(v, 16) == pref16
                b = lax.shift_right_logical(v, 8) & 0xFF
                plsc.addupdate_scatter(
                    hist2_v, [(iota + 16 * half) * 256 + b], ones, mask=m)
                plsc.store_compressed(cbuf_v.at[pl.ds(off, 16)], v, mask=m)
                off = off + jnp.sum(jnp.where(m, 1, 0))
            return off
        cnt = lax.fori_loop(0, NG // 2, _hist2, jnp.int32(0))
        cbuf_v[pl.ds(cnt, 16)] = jnp.full((16,), -1, jnp.int32)
        b, below = _digit(hist_v, k_rem)
        k_rem = k_rem - below
        prefix = prefix | lax.shift_left(b, 8)

    with jax.named_scope("ph_radix3"):
        pref8 = lax.shift_right_logical(prefix, 8)

        def _hist3(i, _):
            v = cbuf_v[pl.ds(i * 16, 16)]
            m = lax.shift_right_logical(v, 8) == pref8
            b = v & 0xFF
            plsc.addupdate_scatter(hist2_v, [iota * 256 + b], ones, mask=m)
            return 0
        ngr3 = lax.div(cnt + 15, jnp.int32(16))
        lax.fori_loop(0, ngr3, _hist3, 0)
        b, below = _digit(hist_v, k_rem)
        k_rem = k_rem - below
        prefix = prefix | b

    t_bits = prefix

    NBLK = 16
    GPB = NG // NBLK
    own_g = w * (NG // NW)
    wlo = w * WIN
    with jax.named_scope("ph_scan"):
        btot = []
        for bi in range(NBLK):
            def _blk(g, acc):
                v = stat_v[pl.ds((bi * GPB + g) * 16, 16)]
                return acc + jnp.where(v <= t_bits, 1, 0)
            acc = lax.fori_loop(0, GPB, _blk, zeros)
            btot.append(jnp.sum(acc))

        bpre = []
        runp = jnp.int32(0)
        for bi in range(NBLK):
            bpre.append(runp)
            runp = runp + btot[bi]

        own_blk = w // 2
        pref_own = jnp.int32(0)
        for bi in range(NBLK):
            pref_own = pref_own + jnp.where(bi < own_blk, btot[bi], 0)

        def _pcnt(g, acc):
            v = stat_v[pl.ds(g * 16, 16)]
            return acc + jnp.sum(jnp.where(v <= t_bits, 1, 0))
        pref_own = lax.fori_loop(own_blk * GPB, own_g, _pcnt, pref_own)

        for bi in range(NBLK):
            hit = (bpre[bi] + btot[bi] > wlo) & (bpre[bi] < wlo + WIN)

            @pl.when(hit)
            def _rescan():
                def _scan(g, run):
                    v = stat_v[pl.ds((bi * GPB + g) * 16, 16)]
                    m = v <= t_bits
                    mi = jnp.where(m, 1, 0)
                    incl = plsc.cumsum(mi)
                    grank = run + incl - mi
                    inwin = m & (grank >= wlo) & (grank < wlo + WIN)
                    plsc.store_scatter(win_v, [grank - wlo],
                                       (bi * GPB + g) * 16 + iota,
                                       mask=inwin)
                    return run + jnp.sum(mi)
                lax.fori_loop(0, GPB, _scan, bpre[bi])

    with jax.named_scope("ph_out"):
        run2 = pref_own
        for g in range(32):
            v = stat_v[pl.ds(w * (B // NW) + g * 16, 16)]
            m = v <= t_bits
            mi = jnp.where(m, 1, 0)
            incl = plsc.cumsum(mi)
            grank = run2 + incl - mi
            run2 = run2 + jnp.sum(mi)
            sel = m & (grank < K)
            posb_v[g // 8, pl.ds((g % 8) * 16, 16)] = jnp.where(sel, grank, -1)
            timeb_v[g // 8, pl.ds((g % 8) * 16, 16)] = jnp.where(sel, 1, 0)

        pltpu.sync_copy(posb_v, pos_hbm.at[pl.ds(w * SEG_ROWS, SEG_ROWS)])
        pltpu.sync_copy(timeb_v, time_hbm.at[pl.ds(w * SEG_ROWS, SEG_ROWS)])

    with jax.named_scope("ph_tail"):
        tbase = jnp.minimum(B // D + w * TAIL_ROWS, PR - TAIL_ROWS)
        pltpu.sync_copy(filln_v, pos_hbm.at[pl.ds(tbase, TAIL_ROWS)])
        pltpu.sync_copy(fillz_v, time_hbm.at[pl.ds(tbase, TAIL_ROWS)])

    with jax.named_scope("ph_gather"):
        for r in range(2):
            for j in range(8):
                win2_v[r, pl.ds(j * 16, 16)] = win_v[pl.ds(r * 128 + j * 16, 16)]
        pltpu.sync_copy(win2_v, lid_hbm.at[pl.ds(2 * w, 2)])
        cp0 = pltpu.async_copy(feats_hbm.at[win2_v.at[0]], rows_v.at[0], sem0)
        cp1 = pltpu.async_copy(feats_hbm.at[win2_v.at[1]], rows_v.at[1], sem1)
        cp0.wait()
        pltpu.sync_copy(rows_v.at[0], emb_hbm.at[pl.ds(wlo, 128)])
        cp1.wait()
        pltpu.sync_copy(rows_v.at[1], emb_hbm.at[pl.ds(wlo + 128, 128)])


def _fused_sc(bits, feats):
    mesh = plsc.VectorSubcoreMesh(core_axis_name="c", subcore_axis_name="s",
                                  num_cores=NC, num_subcores=NS)
    return pl.kernel(
        _fused_body,
        out_type=(
            jax.ShapeDtypeStruct((K, D), jnp.float32),
            jax.ShapeDtypeStruct((K // D, D), jnp.int32),
            jax.ShapeDtypeStruct((PR, D), jnp.int32),
            jax.ShapeDtypeStruct((PR, D), jnp.int32),
        ),
        mesh=mesh,
        compiler_params=pltpu.CompilerParams(needs_layout_passes=False,
                                             use_tc_tiling_on_sc=False),
        scratch_types=[
            pltpu.VMEM((B,), jnp.int32),
            pltpu.VMEM((32 * 256,), jnp.int32),
            pltpu.VMEM((256,), jnp.int32),
            pltpu.VMEM((256,), jnp.int32),
            pltpu.VMEM((B + 16,), jnp.int32),
            pltpu.VMEM((WIN,), jnp.int32),
            pltpu.VMEM((2, 128), jnp.int32),
            pltpu.VMEM((2, 128, D), jnp.float32),
            pltpu.VMEM((SEG_ROWS, D), jnp.int32),
            pltpu.VMEM((SEG_ROWS, D), jnp.int32),
            pltpu.VMEM((TAIL_ROWS, D), jnp.int32),
            pltpu.VMEM((TAIL_ROWS, D), jnp.int32),
            pltpu.SemaphoreType.DMA,
            pltpu.SemaphoreType.DMA,
        ],
    )(bits, feats)


def kernel(gids, feats, grad, pos, time, emb, index_to_gid):
    bits = _stat_tc(grad).reshape(B)
    emb_o, lid2, pos2, time2 = _fused_sc(bits, feats)
    return (emb_o, pos2.reshape(-1)[:N], lid2.reshape(-1),
            time2.reshape(-1)[:N])


# speedup vs baseline: 20.3597x; 1.2950x over previous
import jax
import jax.numpy as jnp
from jax import lax
from jax.experimental import pallas as pl
from jax.experimental.pallas import tpu as pltpu
from jax.experimental.pallas import tpu_sc as plsc

B = 16384
D = 128
N = 100000
K = B // 2
NC = 2
NS = 16
NW = NC * NS
WIN = K // NW
PR = (N + D - 1) // D
SEG_ROWS = B // D // NW
TAIL_ROWS = 21
NG = B // 16


def _stat_tc(grad):
    def body(g_ref, o_ref):
        x = g_ref[...]
        ss = jnp.sum(x * x, axis=1)
        o_ref[...] = lax.bitcast_convert_type(
            jnp.sqrt(ss).reshape(16, 128), jnp.int32)

    return pl.pallas_call(
        body,
        grid=(8,),
        in_specs=[pl.BlockSpec((2048, 128), lambda i: (i, 0))],
        out_specs=pl.BlockSpec((16, 128), lambda i: (i, 0)),
        out_shape=jax.ShapeDtypeStruct((128, 128), jnp.int32),
    )(grad)


def _fused_body(bits_hbm, feats_hbm, emb_hbm, lid_hbm, pos_hbm, time_hbm,
                stat_v, hist2_v, hist_v, cum_v, cbuf_v, win_v, win2_v,
                rows_v, posb_v, timeb_v, filln_v, fillz_v, sem0, sem1):
    c = lax.axis_index("c")
    s = lax.axis_index("s")
    w = c * NS + s
    iota = lax.iota(jnp.int32, 16)
    ones = jnp.full((16,), 1, jnp.int32)
    zeros = jnp.full((16,), 0, jnp.int32)

    with jax.named_scope("ph_fill"):
        @plsc.parallel_loop(0, 512, unroll=8)
        def _zh(i):
            hist2_v[pl.ds(i * 16, 16)] = zeros

        neg1 = jnp.full((16,), -1, jnp.int32)
        for r in range(TAIL_ROWS):
            for j in range(8):
                filln_v[r, pl.ds(j * 16, 16)] = neg1
                fillz_v[r, pl.ds(j * 16, 16)] = zeros

    with jax.named_scope("ph_stage"):
        pltpu.sync_copy(bits_hbm, stat_v)

    def _digit(hist_lo, k_rem):
        def _reduce(g, _):
            acc = hist2_v[pl.ds(g * 16, 16)]
            hist2_v[pl.ds(g * 16, 16)] = zeros
            for r in range(1, 32):
                acc = acc + hist2_v[pl.ds(r * 256 + g * 16, 16)]
                hist2_v[pl.ds(r * 256 + g * 16, 16)] = zeros
            hist_v[pl.ds(g * 16, 16)] = acc
            return 0
        lax.fori_loop(0, 16, _reduce, 0)

        def _cum(g, carry):
            v = hist_v[pl.ds(g * 16, 16)]
            cv = plsc.cumsum(v) + carry
            cum_v[pl.ds(g * 16, 16)] = cv
            return carry + jnp.sum(v)
        lax.fori_loop(0, 16, _cum, jnp.int32(0))

        def _bcnt(g, acc):
            cv = cum_v[pl.ds(g * 16, 16)]
            return acc + jnp.sum(jnp.where(cv < k_rem, 1, 0))
        b = lax.fori_loop(0, 16, _bcnt, jnp.int32(0))

        def _below(g, acc):
            cv = cum_v[pl.ds(g * 16, 16)]
            gl = g * 16 + iota
            return acc + jnp.sum(jnp.where(gl == b - 1, cv, 0))
        below = lax.fori_loop(0, 16, _below, jnp.int32(0))
        return b, below

    prefix = jnp.int32(0)
    k_rem = jnp.int32(K)
    for p in range(2):
      with jax.named_scope(f"ph_radix{p}"):
        shift = 24 - 8 * p

        def _histbody(i, _p=p, _shift=shift, _prefix=prefix):
            for half in range(2):
                v = stat_v[pl.ds((2 * i + half) * 16, 16)]
                b = lax.shift_right_logical(v, _shift) & 0xFF
                if _p == 0:
                    m = None
                else:
                    m = (lax.shift_right_logical(v, _shift + 8) ==
                         lax.shift_right_logical(_prefix, _shift + 8))
                plsc.addupdate_scatter(
                    hist2_v, [(iota + 16 * half) * 256 + b], ones, mask=m)
        plsc.parallel_loop(0, NG // 2, unroll=4)(_histbody)
        b, below = _digit(hist_v, k_rem)
        k_rem = k_rem - below
        prefix = prefix | lax.shift_left(b, shift)

    with jax.named_scope("ph_radix2"):
        pref16 = lax.shift_right_logical(prefix, 16)

        def _hist2(i, off):
            for half in range(2):
                v = stat_v[pl.ds((2 * i + half) * 16, 16)]
                m = lax.shift_right_logical(v, 16) == pref16
                b = lax.shift_right_logical(v, 8) & 0xFF
                plsc.addupdate_scatter(
                    hist2_v, [(iota + 16 * half) * 256 + b], ones, mask=m)
                plsc.store_compressed(cbuf_v.at[pl.ds(off, 16)], v, mask=m)
                off = off + jnp.sum(jnp.where(m, 1, 0))
            return off
        cnt = plsc.parallel_loop(0, NG // 2, unroll=4,
                                 carry=jnp.int32(0))(_hist2)
        cbuf_v[pl.ds(cnt, 16)] = jnp.full((16,), -1, jnp.int32)
        b, below = _digit(hist_v, k_rem)
        k_rem = k_rem - below
        prefix = prefix | lax.shift_left(b, 8)

    with jax.named_scope("ph_radix3"):
        pref8 = lax.shift_right_logical(prefix, 8)

        def _hist3(i, _):
            v = cbuf_v[pl.ds(i * 16, 16)]
            m = lax.shift_right_logical(v, 8) == pref8
            b = v & 0xFF
            plsc.addupdate_scatter(hist2_v, [iota * 256 + b], ones, mask=m)
            return 0
        ngr3 = lax.div(cnt + 15, jnp.int32(16))
        lax.fori_loop(0, ngr3, _hist3, 0)
        b, below = _digit(hist_v, k_rem)
        k_rem = k_rem - below
        prefix = prefix | b

    t_bits = prefix

    NBLK = 16
    GPB = NG // NBLK
    own_g = w * (NG // NW)
    wlo = w * WIN
    with jax.named_scope("ph_scan"):
        btot = []
        for bi in range(NBLK):
            def _blk(g, acc, _bi=bi):
                v = stat_v[pl.ds((_bi * GPB + g) * 16, 16)]
                return acc + jnp.where(v <= t_bits, 1, 0)
            acc = plsc.parallel_loop(0, GPB, unroll=4, carry=zeros)(_blk)
            btot.append(jnp.sum(acc))

        bpre = []
        runp = jnp.int32(0)
        for bi in range(NBLK):
            bpre.append(runp)
            runp = runp + btot[bi]

        own_blk = w // 2
        pref_own = jnp.int32(0)
        for bi in range(NBLK):
            pref_own = pref_own + jnp.where(bi < own_blk, btot[bi], 0)

        def _pcnt(g, acc):
            v = stat_v[pl.ds(g * 16, 16)]
            return acc + jnp.sum(jnp.where(v <= t_bits, 1, 0))
        pref_own = lax.fori_loop(own_blk * GPB, own_g, _pcnt, pref_own)

        for bi in range(NBLK):
            hit = (bpre[bi] + btot[bi] > wlo) & (bpre[bi] < wlo + WIN)

            @pl.when(hit)
            def _rescan():
                def _scan(g, run):
                    v = stat_v[pl.ds((bi * GPB + g) * 16, 16)]
                    m = v <= t_bits
                    mi = jnp.where(m, 1, 0)
                    incl = plsc.cumsum(mi)
                    grank = run + incl - mi
                    inwin = m & (grank >= wlo) & (grank < wlo + WIN)
                    plsc.store_scatter(win_v, [grank - wlo],
                                       (bi * GPB + g) * 16 + iota,
                                       mask=inwin)
                    return run + jnp.sum(mi)
                lax.fori_loop(0, GPB, _scan, bpre[bi])

    with jax.named_scope("ph_out"):
        run2 = pref_own
        for g in range(32):
            v = stat_v[pl.ds(w * (B // NW) + g * 16, 16)]
            m = v <= t_bits
            mi = jnp.where(m, 1, 0)
            incl = plsc.cumsum(mi)
            grank = run2 + incl - mi
            run2 = run2 + jnp.sum(mi)
            sel = m & (grank < K)
            posb_v[g // 8, pl.ds((g % 8) * 16, 16)] = jnp.where(sel, grank, -1)
            timeb_v[g // 8, pl.ds((g % 8) * 16, 16)] = jnp.where(sel, 1, 0)

        pltpu.sync_copy(posb_v, pos_hbm.at[pl.ds(w * SEG_ROWS, SEG_ROWS)])
        pltpu.sync_copy(timeb_v, time_hbm.at[pl.ds(w * SEG_ROWS, SEG_ROWS)])

    with jax.named_scope("ph_tail"):
        tbase = jnp.minimum(B // D + w * TAIL_ROWS, PR - TAIL_ROWS)
        pltpu.sync_copy(filln_v, pos_hbm.at[pl.ds(tbase, TAIL_ROWS)])
        pltpu.sync_copy(fillz_v, time_hbm.at[pl.ds(tbase, TAIL_ROWS)])

    with jax.named_scope("ph_gather"):
        for r in range(2):
            for j in range(8):
                win2_v[r, pl.ds(j * 16, 16)] = win_v[pl.ds(r * 128 + j * 16, 16)]
        pltpu.sync_copy(win2_v, lid_hbm.at[pl.ds(2 * w, 2)])
        cp0 = pltpu.async_copy(feats_hbm.at[win2_v.at[0]], rows_v.at[0], sem0)
        cp1 = pltpu.async_copy(feats_hbm.at[win2_v.at[1]], rows_v.at[1], sem1)
        cp0.wait()
        pltpu.sync_copy(rows_v.at[0], emb_hbm.at[pl.ds(wlo, 128)])
        cp1.wait()
        pltpu.sync_copy(rows_v.at[1], emb_hbm.at[pl.ds(wlo + 128, 128)])


def _fused_sc(bits, feats):
    mesh = plsc.VectorSubcoreMesh(core_axis_name="c", subcore_axis_name="s",
                                  num_cores=NC, num_subcores=NS)
    return pl.kernel(
        _fused_body,
        out_type=(
            jax.ShapeDtypeStruct((K, D), jnp.float32),
            jax.ShapeDtypeStruct((K // D, D), jnp.int32),
            jax.ShapeDtypeStruct((PR, D), jnp.int32),
            jax.ShapeDtypeStruct((PR, D), jnp.int32),
        ),
        mesh=mesh,
        compiler_params=pltpu.CompilerParams(needs_layout_passes=False,
                                             use_tc_tiling_on_sc=False),
        scratch_types=[
            pltpu.VMEM((B,), jnp.int32),
            pltpu.VMEM((32 * 256,), jnp.int32),
            pltpu.VMEM((256,), jnp.int32),
            pltpu.VMEM((256,), jnp.int32),
            pltpu.VMEM((B + 16,), jnp.int32),
            pltpu.VMEM((WIN,), jnp.int32),
            pltpu.VMEM((2, 128), jnp.int32),
            pltpu.VMEM((2, 128, D), jnp.float32),
            pltpu.VMEM((SEG_ROWS, D), jnp.int32),
            pltpu.VMEM((SEG_ROWS, D), jnp.int32),
            pltpu.VMEM((TAIL_ROWS, D), jnp.int32),
            pltpu.VMEM((TAIL_ROWS, D), jnp.int32),
            pltpu.SemaphoreType.DMA,
            pltpu.SemaphoreType.DMA,
        ],
    )(bits, feats)


def kernel(gids, feats, grad, pos, time, emb, index_to_gid):
    bits = _stat_tc(grad).reshape(B)
    emb_o, lid2, pos2, time2 = _fused_sc(bits, feats)
    return (emb_o, pos2.reshape(-1)[:N], lid2.reshape(-1),
            time2.reshape(-1)[:N])
